# baseline (speedup 1.0000x reference)
import functools

import jax
import jax.numpy as jnp
from jax.experimental import pallas as pl
from jax.experimental.pallas import tpu as pltpu
from jax.experimental.pallas import tpu_sc as plsc

_W = 8


def _lookup_body(y_hbm, table_hbm, out_hbm, idx_v, rows_v, sem):
    @pl.when(jax.lax.axis_index("s") == 0)
    def _():
        pltpu.sync_copy(y_hbm, idx_v)
        pltpu.async_copy(table_hbm.at[idx_v], rows_v, sem).wait()
        pltpu.sync_copy(rows_v, out_hbm)


_mesh = plsc.VectorSubcoreMesh(
    core_axis_name="c", subcore_axis_name="s", num_cores=1
)

_lookup = functools.partial(
    pl.kernel,
    mesh=_mesh,
    out_type=jax.ShapeDtypeStruct((_W,), jnp.int32),
    scratch_types=[
        pltpu.VMEM((_W,), jnp.int32),
        pltpu.VMEM((_W,), jnp.int32),
        pltpu.SemaphoreType.DMA,
    ],
)(_lookup_body)


def kernel(y, label_permutation):
    out_dtype = label_permutation.dtype
    table32 = label_permutation.astype(jnp.int32)
    y32 = jnp.full((_W,), y, jnp.int32)
    out = _lookup(y32, table32)
    return out[0].astype(out_dtype)


# speedup vs baseline: 1.0023x; 1.0023x over previous
import functools

import jax
import jax.numpy as jnp
from jax.experimental import pallas as pl
from jax.experimental.pallas import tpu as pltpu
from jax.experimental.pallas import tpu_sc as plsc

_W = 8


def _lookup_body(y_hbm, table_hbm, out_hbm, idx_v, rows_v, sem):
    @pl.when(jax.lax.axis_index("s") == 0)
    def _():
        pltpu.sync_copy(y_hbm, idx_v)
        pltpu.async_copy(table_hbm.at[idx_v], rows_v, sem).wait()
        pltpu.sync_copy(rows_v, out_hbm)


_mesh = plsc.VectorSubcoreMesh(
    core_axis_name="c", subcore_axis_name="s", num_cores=1
)

_lookup = functools.partial(
    pl.kernel,
    mesh=_mesh,
    out_type=jax.ShapeDtypeStruct((_W,), jnp.int32),
    compiler_params=pltpu.CompilerParams(skip_device_barrier=True),
    scratch_types=[
        pltpu.VMEM((_W,), jnp.int32),
        pltpu.VMEM((_W,), jnp.int32),
        pltpu.SemaphoreType.DMA,
    ],
)(_lookup_body)


def kernel(y, label_permutation):
    out_dtype = label_permutation.dtype
    table32 = label_permutation.astype(jnp.int32)
    y32 = jnp.full((_W,), y, jnp.int32)
    out = _lookup(y32, table32)
    return out[0].astype(out_dtype)
